# baseline (speedup 1.0000x reference)
import functools

import jax
import jax.numpy as jnp
from jax import lax
from jax.experimental import pallas as pl
from jax.experimental.pallas import tpu as pltpu
from jax.experimental.pallas import tpu_sc as plsc

N = 8192
D = 64
KC = 8192

BN = 256
NLANE = 128

CW = 16


BKC = 1024


def _argmin_body(flat_ref, w_ref, f2_ref, w2_ref, idx_ref, mind_ref):
    flatb = flat_ref[...]
    f2 = f2_ref[...]
    lane = lax.broadcasted_iota(jnp.int32, (BN, NLANE), 1)
    v = jnp.full((BN, NLANE), jnp.inf, jnp.float32)
    a = jnp.zeros((BN, NLANE), jnp.int32)
    for c in range(KC // BKC):
        wb = w_ref[pl.ds(c * BKC, BKC), :]
        mm = lax.dot_general(flatb, wb, (((1,), (1,)), ((), ())),
                             preferred_element_type=jnp.float32)
        for t in range(BKC // NLANE):
            s = c * (BKC // NLANE) + t
            mmc = lax.slice(mm, (0, t * NLANE), (BN, (t + 1) * NLANE))
            w2c = lax.slice(w2_ref[...], (0, s * NLANE), (1, (s + 1) * NLANE))
            d = (f2 + w2c) - 2.0 * mmc
            m = d < v
            v = jnp.where(m, d, v)
            a = jnp.where(m, lane + s * NLANE, a)
    lmin = jnp.min(v, axis=1, keepdims=True)
    cand = jnp.where(v == lmin, a, KC)
    idx_ref[...] = jnp.min(cand, axis=1, keepdims=True)
    mind_ref[...] = lmin


def _argmin_call(flat, W, f2, w2):
    return pl.pallas_call(
        _argmin_body,
        grid=(N // BN,),
        in_specs=[
            pl.BlockSpec((BN, D), lambda i: (i, 0)),
            pl.BlockSpec((KC, D), lambda i: (0, 0)),
            pl.BlockSpec((BN, 1), lambda i: (i, 0)),
            pl.BlockSpec((1, KC), lambda i: (0, 0)),
        ],
        out_specs=[
            pl.BlockSpec((BN, 1), lambda i: (i, 0)),
            pl.BlockSpec((BN, 1), lambda i: (i, 0)),
        ],
        out_shape=[
            jax.ShapeDtypeStruct((N, 1), jnp.int32),
            jax.ShapeDtypeStruct((N, 1), jnp.float32),
        ],
        compiler_params=pltpu.CompilerParams(
            dimension_semantics=("parallel",)),
    )(flat, W, f2, w2)


def _stats_body(cnt_ref, mind_ref, loss_ref, perp_ref):
    p = cnt_ref[...] * (1.0 / N)
    ent = jnp.sum(p * jnp.log(p + 1e-10))
    perp_ref[...] = jnp.exp(-ent) * jnp.ones((1, 1), jnp.float32)
    loss_ref[...] = (jnp.sum(mind_ref[...]) * (1.25 / (N * D))
                     * jnp.ones((1, 1), jnp.float32))


def _stats_call(cnt2, mind2):
    return pl.pallas_call(
        _stats_body,
        grid=(1,),
        in_specs=[
            pl.BlockSpec((KC // 128, 128), lambda b: (0, 0)),
            pl.BlockSpec((N // 128, 128), lambda b: (0, 0)),
        ],
        out_specs=[
            pl.BlockSpec((1, 1), lambda b: (0, 0)),
            pl.BlockSpec((1, 1), lambda b: (0, 0)),
        ],
        out_shape=[
            jax.ShapeDtypeStruct((1, 1), jnp.float32),
            jax.ShapeDtypeStruct((1, 1), jnp.float32),
        ],
    )(cnt2, mind2)


@functools.lru_cache(maxsize=1)
def _make_sc_gather():
    info = plsc.get_sparse_core_info()
    nc, ns = info.num_cores, info.num_subcores
    nw = nc * ns
    bpw = N // nw
    chunks = bpw // 128
    kpw = KC // ns
    mesh = plsc.VectorSubcoreMesh(core_axis_name="c", subcore_axis_name="s")

    @functools.partial(
        pl.kernel, mesh=mesh,
        out_type=[
            jax.ShapeDtypeStruct((N, D), jnp.float32),
            jax.ShapeDtypeStruct((nc, KC, CW), jnp.float32),
        ],
        scratch_types=[
            pltpu.VMEM((chunks, 128), jnp.int32),
            pltpu.VMEM((bpw, D), jnp.float32),
            pltpu.VMEM((128, CW), jnp.float32),
            pltpu.VMEM_SHARED((KC, CW), jnp.float32),
            pltpu.SemaphoreType.DMA,
        ],
        compiler_params=pltpu.CompilerParams(use_tc_tiling_on_sc=False),
    )
    def gather_kernel(table_hbm, idx_hbm, zeros_hbm, ones_hbm,
                      out_hbm, cnt_hbm, idx_v, rows_v, ones_v, cshared, sem):
        cid = lax.axis_index("c")
        sid = lax.axis_index("s")
        wid = sid * nc + cid
        pltpu.sync_copy(idx_hbm.at[pl.ds(wid * chunks, chunks)], idx_v)
        cps = [
            pltpu.async_copy(table_hbm.at[idx_v.at[c]],
                             rows_v.at[pl.ds(c * 128, 128)], sem)
            for c in range(chunks)
        ]
        pltpu.sync_copy(ones_hbm, ones_v)
        pltpu.sync_copy(zeros_hbm.at[pl.ds(sid * kpw, kpw)],
                        cshared.at[pl.ds(sid * kpw, kpw)])
        plsc.subcore_barrier()
        for c in range(chunks):
            pltpu.sync_copy(ones_v, cshared.at[idx_v.at[c]], add=True)
        plsc.subcore_barrier()
        pltpu.sync_copy(cshared.at[pl.ds(sid * kpw, kpw)],
                        cnt_hbm.at[cid, pl.ds(sid * kpw, kpw)])
        for cp in cps:
            cp.wait()
        pltpu.sync_copy(rows_v, out_hbm.at[pl.ds(wid * bpw, bpw)])

    return gather_kernel


def kernel(z, W):
    B, C, H, Wd = z.shape
    zt = jnp.transpose(z, (0, 2, 3, 1))
    flat = zt.reshape(N, D)
    f2 = jnp.sum(flat ** 2, axis=1, keepdims=True)
    w2 = jnp.sum(W ** 2, axis=1).reshape(1, KC)

    idx2d, mind = _argmin_call(flat, W, f2, w2)

    idx_rows = idx2d.reshape(N // 128, 128)
    zeros = jnp.zeros((KC, CW), jnp.float32)
    ones = jnp.ones((128, CW), jnp.float32)
    z_q, cnt = _make_sc_gather()(W, idx_rows, zeros, ones)

    cnt2 = (cnt[0, :, 0] + cnt[1, :, 0]).reshape(KC // 128, 128)
    mind2 = mind.reshape(N // 128, 128)
    loss2d, perp2d = _stats_call(cnt2, mind2)

    out = jnp.transpose(z_q.reshape(B, H, Wd, C), (0, 3, 1, 2))
    loss = loss2d.reshape(())
    perplexity = perp2d.reshape(())
    min_encoding_indices = idx2d.reshape(B, H, Wd)
    return (out, loss, min_encoding_indices, perplexity)


# speedup vs baseline: 1.0560x; 1.0560x over previous
import functools

import jax
import jax.numpy as jnp
from jax import lax
from jax.experimental import pallas as pl
from jax.experimental.pallas import tpu as pltpu
from jax.experimental.pallas import tpu_sc as plsc

N = 8192
D = 64
KC = 8192

BN = 256
NLANE = 128

CW = 16


BKC = 1024


def _argmin_body(flat_ref, w_ref, f2_ref, w2_ref, idx_ref, mind_ref):
    flatb = flat_ref[...]
    f2 = f2_ref[...]
    lane = lax.broadcasted_iota(jnp.int32, (BN, NLANE), 1)
    v = jnp.full((BN, NLANE), jnp.inf, jnp.float32)
    a = jnp.zeros((BN, NLANE), jnp.int32)
    for c in range(KC // BKC):
        wb = w_ref[pl.ds(c * BKC, BKC), :]
        mm = lax.dot_general(flatb, wb, (((1,), (1,)), ((), ())),
                             preferred_element_type=jnp.float32)
        for t in range(BKC // NLANE):
            s = c * (BKC // NLANE) + t
            mmc = lax.slice(mm, (0, t * NLANE), (BN, (t + 1) * NLANE))
            w2c = lax.slice(w2_ref[...], (0, s * NLANE), (1, (s + 1) * NLANE))
            d = (f2 + w2c) - 2.0 * mmc
            m = d < v
            v = jnp.where(m, d, v)
            a = jnp.where(m, lane + s * NLANE, a)
    lmin = jnp.min(v, axis=1, keepdims=True)
    cand = jnp.where(v == lmin, a, KC)
    idx_ref[...] = jnp.min(cand, axis=1, keepdims=True)
    mind_ref[...] = lmin


def _argmin_call(flat, W, f2, w2):
    return pl.pallas_call(
        _argmin_body,
        grid=(N // BN,),
        in_specs=[
            pl.BlockSpec((BN, D), lambda i: (i, 0)),
            pl.BlockSpec((KC, D), lambda i: (0, 0)),
            pl.BlockSpec((BN, 1), lambda i: (i, 0)),
            pl.BlockSpec((1, KC), lambda i: (0, 0)),
        ],
        out_specs=[
            pl.BlockSpec((BN, 1), lambda i: (i, 0)),
            pl.BlockSpec((BN, 1), lambda i: (i, 0)),
        ],
        out_shape=[
            jax.ShapeDtypeStruct((N, 1), jnp.int32),
            jax.ShapeDtypeStruct((N, 1), jnp.float32),
        ],
        compiler_params=pltpu.CompilerParams(
            dimension_semantics=("parallel",)),
    )(flat, W, f2, w2)


def _stats_body(cnt_ref, mind_ref, loss_ref, perp_ref):
    p = cnt_ref[...] * (1.0 / N)
    ent = jnp.sum(p * jnp.log(p + 1e-10))
    perp_ref[...] = jnp.exp(-ent) * jnp.ones((1, 1), jnp.float32)
    loss_ref[...] = (jnp.sum(mind_ref[...]) * (1.25 / (N * D))
                     * jnp.ones((1, 1), jnp.float32))


def _stats_call(cnt2, mind2):
    return pl.pallas_call(
        _stats_body,
        grid=(1,),
        in_specs=[
            pl.BlockSpec((KC // 128, 128), lambda b: (0, 0)),
            pl.BlockSpec((N // 128, 128), lambda b: (0, 0)),
        ],
        out_specs=[
            pl.BlockSpec((1, 1), lambda b: (0, 0)),
            pl.BlockSpec((1, 1), lambda b: (0, 0)),
        ],
        out_shape=[
            jax.ShapeDtypeStruct((1, 1), jnp.float32),
            jax.ShapeDtypeStruct((1, 1), jnp.float32),
        ],
    )(cnt2, mind2)


@functools.lru_cache(maxsize=1)
def _make_sc_gather():
    info = plsc.get_sparse_core_info()
    nc, ns = info.num_cores, info.num_subcores
    nw = nc * ns
    bpw = N // nw
    chunks = bpw // 128
    kpw = KC // ns
    mesh = plsc.VectorSubcoreMesh(core_axis_name="c", subcore_axis_name="s")

    @functools.partial(
        pl.kernel, mesh=mesh,
        out_type=[
            jax.ShapeDtypeStruct((N, D), jnp.float32),
            jax.ShapeDtypeStruct((nc, KC), jnp.float32),
        ],
        scratch_types=[
            pltpu.VMEM((chunks, 128), jnp.int32),
            pltpu.VMEM((bpw, D), jnp.float32),
            pltpu.VMEM((128, CW), jnp.float32),
            pltpu.VMEM((kpw, CW), jnp.float32),
            pltpu.VMEM((kpw,), jnp.float32),
            pltpu.VMEM_SHARED((KC, CW), jnp.float32),
            pltpu.SemaphoreType.DMA,
        ],
        compiler_params=pltpu.CompilerParams(use_tc_tiling_on_sc=False,
                                             needs_layout_passes=False),
    )
    def gather_kernel(table_hbm, idx_hbm, zeros_hbm, ones_hbm,
                      out_hbm, cnt_hbm, idx_v, rows_v, ones_v, craw_v, cvec_v,
                      cshared, sem):
        cid = lax.axis_index("c")
        sid = lax.axis_index("s")
        wid = sid * nc + cid
        pltpu.sync_copy(idx_hbm.at[pl.ds(wid * chunks, chunks)], idx_v)
        cps = [
            pltpu.async_copy(table_hbm.at[idx_v.at[c]],
                             rows_v.at[pl.ds(c * 128, 128)], sem)
            for c in range(chunks)
        ]
        pltpu.sync_copy(ones_hbm, ones_v)
        pltpu.sync_copy(zeros_hbm.at[pl.ds(sid * kpw, kpw)],
                        cshared.at[pl.ds(sid * kpw, kpw)])
        plsc.subcore_barrier()
        for c in range(chunks):
            pltpu.sync_copy(ones_v, cshared.at[idx_v.at[c]], add=True)
        plsc.subcore_barrier()
        pltpu.sync_copy(cshared.at[pl.ds(sid * kpw, kpw)], craw_v)
        row16 = lax.iota(jnp.int32, 16)
        col16 = jnp.zeros((16,), jnp.int32)
        for g in range(kpw // 16):
            vals = plsc.load_gather(craw_v, [row16 + g * 16, col16])
            cvec_v[pl.ds(g * 16, 16)] = vals
        pltpu.sync_copy(cvec_v, cnt_hbm.at[cid, pl.ds(sid * kpw, kpw)])
        for cp in cps:
            cp.wait()
        pltpu.sync_copy(rows_v, out_hbm.at[pl.ds(wid * bpw, bpw)])

    return gather_kernel


def kernel(z, W):
    B, C, H, Wd = z.shape
    zt = jnp.transpose(z, (0, 2, 3, 1))
    flat = zt.reshape(N, D)
    f2 = jnp.sum(flat ** 2, axis=1, keepdims=True)
    w2 = jnp.sum(W ** 2, axis=1).reshape(1, KC)

    idx2d, mind = _argmin_call(flat, W, f2, w2)

    idx_rows = idx2d.reshape(N // 128, 128)
    zeros = jnp.zeros((KC, CW), jnp.float32)
    ones = jnp.ones((128, CW), jnp.float32)
    z_q, cnt = _make_sc_gather()(W, idx_rows, zeros, ones)

    cnt2 = (cnt[0] + cnt[1]).reshape(KC // 128, 128)
    mind2 = mind.reshape(N // 128, 128)
    loss2d, perp2d = _stats_call(cnt2, mind2)

    out = jnp.transpose(z_q.reshape(B, H, Wd, C), (0, 3, 1, 2))
    loss = loss2d.reshape(())
    perplexity = perp2d.reshape(())
    min_encoding_indices = idx2d.reshape(B, H, Wd)
    return (out, loss, min_encoding_indices, perplexity)
